# baseline (speedup 1.0000x reference)
import jax
import jax.numpy as jnp
from jax import lax
from jax.experimental import pallas as pl
from jax.experimental.pallas import tpu as pltpu
from jax.experimental.pallas import tpu_sc as plsc

KNN = 12
RT = 448
TCH = 448
BIG = 1e9

NW = 32
CH = 56
NL = 16


def _t0(a, b):
    return lax.dot_general(a, b, (((0,), (0,)), ((), ())),
                           preferred_element_type=jnp.float32)


def _fc1_kernel(x_ref, w_ref, br_ref, bc_ref, h_ref, ht_ref, sq_ref):
    xb = x_ref[0]
    w = w_ref[...]
    h_ref[0] = _t0(xb, w) + br_ref[...]
    htb = _t0(w, xb) + bc_ref[...]
    ht_ref[0] = htb
    sq_ref[0] = jnp.sum(htb * htb, axis=0, keepdims=True)


def _knn_kernel(hr_ref, ht_ref, sq_ref, idx_ref):
    hb = hr_ref[0]
    ht = ht_ref[0]
    n = ht.shape[1]
    d = sq_ref[0] - 2.0 * jnp.dot(hb, ht, preferred_element_type=jnp.float32)
    ii = jax.lax.broadcasted_iota(jnp.int32, (RT, n), 1).astype(jnp.float32)
    cols = []
    for t in range(KNN):
        m = jnp.min(d, axis=1, keepdims=True)
        cand = jnp.where(d == m, ii, BIG)
        sel = jnp.min(cand, axis=1, keepdims=True)
        cols.append(sel.astype(jnp.int32))
        if t < KNN - 1:
            d = jnp.where(cand == sel, BIG, d)
    idx_ref[0] = jnp.concatenate(cols, axis=1)


def _make_sc_gather_max(npw, n_per_b, n_active):
    nchunk = npw // CH
    groups = npw // NL

    def _sc_gather_max(h_hbm, idx_hbm, out_hbm, idx_v, idxt_v, rows_v, out_v,
                       sem0, sem1):
        c_dim = h_hbm.shape[1]
        wid = lax.axis_index("s") * 2 + lax.axis_index("c")
        base = wid * npw

        @pl.when(wid < n_active)
        def _():
            pltpu.sync_copy(idx_hbm.at[pl.ds(base, npw)], idx_v)
            boff = (base // n_per_b) * n_per_b
            lanes = lax.iota(jnp.int32, NL)
            for j in range(KNN):
                col = jnp.full((NL,), j, jnp.int32)
                for g in range(groups):
                    rows = lanes + g * NL
                    vals = plsc.load_gather(idx_v, [rows, col])
                    idxt_v[j, pl.ds(g * NL, NL)] = vals + boff
            sems = (sem0, sem1)

            def fire(c):
                cps = []
                for j in range(KNN):
                    cps.append(pltpu.async_copy(
                        h_hbm.at[idxt_v.at[j, pl.ds(c * CH, CH)]],
                        rows_v.at[c % 2, j], sems[c % 2]))
                return cps

            cps = fire(0)
            for c in range(nchunk):
                nxt = fire(c + 1) if c + 1 < nchunk else None
                for cp in cps:
                    cp.wait()
                buf = c % 2

                def body(nn, _):
                    for dd in range(c_dim // NL):
                        acc = rows_v[buf, 0, nn, pl.ds(dd * NL, NL)]
                        for j in range(1, KNN):
                            acc = jnp.maximum(
                                acc, rows_v[buf, j, nn, pl.ds(dd * NL, NL)])
                        out_v[nn, pl.ds(dd * NL, NL)] = acc
                    return 0

                lax.fori_loop(0, CH, body, 0)
                pltpu.sync_copy(out_v, out_hbm.at[pl.ds(base + c * CH, CH)])
                cps = nxt

    return _sc_gather_max


def _tail_kernel(x_ref, ht_ref, mh_ref, wga_ref, wgb_ref, bg_ref,
                 w2_ref, b2_ref, wf1_ref, bf1_ref, wf2_ref, bf2_ref, y_ref):
    xb = x_ref[0]
    htb = ht_ref[0]
    mh = mh_ref[0]
    n = xb.shape[1]
    eye = (jax.lax.broadcasted_iota(jnp.int32, (TCH, TCH), 0)
           == jax.lax.broadcasted_iota(jnp.int32, (TCH, TCH), 1)
           ).astype(jnp.float32)
    mht = jnp.concatenate(
        [_t0(mh[k * TCH:(k + 1) * TCH], eye) for k in range(n // TCH)],
        axis=1)
    mrt = mht - htb
    gt = (
        jnp.dot(wga_ref[...], htb, preferred_element_type=jnp.float32)
        + jnp.dot(wgb_ref[...], mrt, preferred_element_type=jnp.float32)
        + bg_ref[...]
    )
    gt = jax.nn.gelu(gt)
    outt = (
        jnp.dot(w2_ref[...], gt, preferred_element_type=jnp.float32)
        + b2_ref[...] + xb
    )
    ft = jax.nn.gelu(
        jnp.dot(wf1_ref[...], outt, preferred_element_type=jnp.float32)
        + bf1_ref[...]
    )
    y_ref[0] = (
        jnp.dot(wf2_ref[...], ft, preferred_element_type=jnp.float32)
        + bf2_ref[...] + outt
    )


def kernel(x, W_fc1, b_fc1, g1, be1, W_g, b_g, gg, bg, W_fc2, b_fc2, g2, be2,
           Wf1, bf1, gf1, bef1, Wf2, bf2, gf2, bef2):
    B, C, H, W = x.shape
    N = H * W
    BN = B * N

    W1p = W_fc1 * g1[None, :]
    b1r = (b_fc1 * g1 + be1)[None, :]
    b1c = (b_fc1 * g1 + be1)[:, None]
    Wgp = W_g * gg[None, :]
    bgc = (b_g * gg + bg)[:, None]
    WgAT, WgBT = Wgp[:C].T, Wgp[C:].T
    W2T = (W_fc2 * g2[None, :]).T
    b2c = (b_fc2 * g2 + be2)[:, None]
    Wf1T = (Wf1 * gf1[None, :]).T
    bf1c = (bf1 * gf1 + bef1)[:, None]
    Wf2T = (Wf2 * gf2[None, :]).T
    bf2c = (bf2 * gf2 + bef2)[:, None]

    x3 = x.reshape(B, C, N)

    h, ht, sq = pl.pallas_call(
        _fc1_kernel,
        grid=(B,),
        in_specs=[
            pl.BlockSpec((1, C, N), lambda b: (b, 0, 0)),
            pl.BlockSpec((C, C), lambda b: (0, 0)),
            pl.BlockSpec((1, C), lambda b: (0, 0)),
            pl.BlockSpec((C, 1), lambda b: (0, 0)),
        ],
        out_specs=[
            pl.BlockSpec((1, N, C), lambda b: (b, 0, 0)),
            pl.BlockSpec((1, C, N), lambda b: (b, 0, 0)),
            pl.BlockSpec((1, 1, N), lambda b: (b, 0, 0)),
        ],
        out_shape=[
            jax.ShapeDtypeStruct((B, N, C), jnp.float32),
            jax.ShapeDtypeStruct((B, C, N), jnp.float32),
            jax.ShapeDtypeStruct((B, 1, N), jnp.float32),
        ],
    )(x3, W1p, b1r, b1c)

    mesh = plsc.VectorSubcoreMesh(core_axis_name="c", subcore_axis_name="s")
    nb = B // 2
    npw = 224
    n_active = (nb * N) // npw
    groups = npw // NL
    sc_body = _make_sc_gather_max(npw, N, n_active)
    mh_halves = []
    for s in range(2):
        hs = h[s * nb:(s + 1) * nb]
        hts = ht[s * nb:(s + 1) * nb]
        sqs = sq[s * nb:(s + 1) * nb]
        idx_s = pl.pallas_call(
            _knn_kernel,
            grid=(nb, N // RT),
            in_specs=[
                pl.BlockSpec((1, RT, C), lambda b, r: (b, r, 0)),
                pl.BlockSpec((1, C, N), lambda b, r: (b, 0, 0)),
                pl.BlockSpec((1, 1, N), lambda b, r: (b, 0, 0)),
            ],
            out_specs=pl.BlockSpec((1, RT, KNN), lambda b, r: (b, r, 0)),
            out_shape=jax.ShapeDtypeStruct((nb, N, KNN), jnp.int32),
        )(hs, hts, sqs)
        mh = pl.kernel(
            sc_body,
            out_type=jax.ShapeDtypeStruct((nb * N, C), jnp.float32),
            mesh=mesh,
            scratch_types=[
                pltpu.VMEM((npw, KNN), jnp.int32),
                pltpu.VMEM((KNN, npw), jnp.int32),
                pltpu.VMEM((2, KNN, CH, C), jnp.float32),
                pltpu.VMEM((CH, C), jnp.float32),
                pltpu.SemaphoreType.DMA,
                pltpu.SemaphoreType.DMA,
            ],
            compiler_params=pltpu.CompilerParams(
                use_tc_tiling_on_sc=False, needs_layout_passes=False),
        )(hs.reshape(nb * N, C), idx_s.reshape(nb * N, KNN))
        mh_halves.append(mh)
    maxh_rows = jnp.concatenate(mh_halves, axis=0)

    y = pl.pallas_call(
        _tail_kernel,
        grid=(B,),
        in_specs=[
            pl.BlockSpec((1, C, N), lambda b: (b, 0, 0)),
            pl.BlockSpec((1, C, N), lambda b: (b, 0, 0)),
            pl.BlockSpec((1, N, C), lambda b: (b, 0, 0)),
            pl.BlockSpec((2 * C, C), lambda b: (0, 0)),
            pl.BlockSpec((2 * C, C), lambda b: (0, 0)),
            pl.BlockSpec((2 * C, 1), lambda b: (0, 0)),
            pl.BlockSpec((C, 2 * C), lambda b: (0, 0)),
            pl.BlockSpec((C, 1), lambda b: (0, 0)),
            pl.BlockSpec((4 * C, C), lambda b: (0, 0)),
            pl.BlockSpec((4 * C, 1), lambda b: (0, 0)),
            pl.BlockSpec((C, 4 * C), lambda b: (0, 0)),
            pl.BlockSpec((C, 1), lambda b: (0, 0)),
        ],
        out_specs=pl.BlockSpec((1, C, N), lambda b: (b, 0, 0)),
        out_shape=jax.ShapeDtypeStruct((B, C, N), jnp.float32),
    )(x3, ht, maxh_rows.reshape(B, N, C), WgAT, WgBT, bgc, W2T, b2c,
      Wf1T, bf1c, Wf2T, bf2c)

    return y.reshape(B, C, H, W)


# speedup vs baseline: 1.0094x; 1.0094x over previous
import jax
import jax.numpy as jnp
from jax import lax
from jax.experimental import pallas as pl
from jax.experimental.pallas import tpu as pltpu
from jax.experimental.pallas import tpu_sc as plsc

KNN = 12
RT = 448
TCH = 448
BIG = 1e9

NW = 32
CH = 56
NL = 16


def _t0(a, b):
    return lax.dot_general(a, b, (((0,), (0,)), ((), ())),
                           preferred_element_type=jnp.float32)


def _fc1_kernel(x_ref, w_ref, br_ref, bc_ref, h_ref, ht_ref, sq_ref):
    xb = x_ref[0]
    w = w_ref[...]
    h_ref[0] = _t0(xb, w) + br_ref[...]
    htb = _t0(w, xb) + bc_ref[...]
    ht_ref[0] = htb
    sq_ref[0] = jnp.sum(htb * htb, axis=0, keepdims=True)


def _knn_kernel(hr_ref, ht_ref, sq_ref, idx_ref):
    hb = hr_ref[0]
    ht = ht_ref[0]
    n = ht.shape[1]
    d = sq_ref[0] - 2.0 * jnp.dot(hb, ht, preferred_element_type=jnp.float32)
    ii = jax.lax.broadcasted_iota(jnp.int32, (RT, n), 1).astype(jnp.float32)
    cols = []
    for t in range(KNN):
        m = jnp.min(d, axis=1, keepdims=True)
        cand = jnp.where(d == m, ii, BIG)
        sel = jnp.min(cand, axis=1, keepdims=True)
        cols.append(sel.astype(jnp.int32))
        if t < KNN - 1:
            d = jnp.where(cand == sel, BIG, d)
    idx_ref[0] = jnp.concatenate(cols, axis=1)


def _sc_gather_max(h_hbm, idx_hbm, out_hbm, idx_v, idxt_v, rows_v, out_v,
                   sem0, sem1):
    nodes_pw = 392
    nchunk = nodes_pw // CH
    c_dim = h_hbm.shape[1]
    n_per_b = idx_hbm.shape[0] // 4
    wid = lax.axis_index("s") * 2 + lax.axis_index("c")
    base = wid * nodes_pw
    pltpu.sync_copy(idx_hbm.at[pl.ds(base, nodes_pw)],
                    idx_v.at[pl.ds(0, nodes_pw)])
    boff = (wid // (NW // 4)) * n_per_b
    lanes = lax.iota(jnp.int32, NL)
    for j in range(KNN):
        col = jnp.full((NL,), j, jnp.int32)
        for g in range(25):
            rows = lanes + g * NL
            vals = plsc.load_gather(idx_v, [rows, col])
            idxt_v[j, pl.ds(g * NL, NL)] = vals + boff
    sems = (sem0, sem1)

    def fire(c):
        cps = []
        for j in range(KNN):
            cps.append(pltpu.async_copy(
                h_hbm.at[idxt_v.at[j, pl.ds(c * CH, CH)]],
                rows_v.at[c % 2, j], sems[c % 2]))
        return cps

    cps = fire(0)
    for c in range(nchunk):
        nxt = fire(c + 1) if c + 1 < nchunk else None
        for cp in cps:
            cp.wait()
        buf = c % 2

        def body(nn, _):
            for dd in range(c_dim // NL):
                acc = rows_v[buf, 0, nn, pl.ds(dd * NL, NL)]
                for j in range(1, KNN):
                    acc = jnp.maximum(
                        acc, rows_v[buf, j, nn, pl.ds(dd * NL, NL)])
                out_v[nn, pl.ds(dd * NL, NL)] = acc
            return 0

        lax.fori_loop(0, CH, body, 0)
        pltpu.sync_copy(out_v, out_hbm.at[pl.ds(base + c * CH, CH)])
        cps = nxt


def _tail_kernel(x_ref, ht_ref, mh_ref, wga_ref, wgb_ref, bg_ref,
                 w2_ref, b2_ref, wf1_ref, bf1_ref, wf2_ref, bf2_ref, y_ref):
    xb = x_ref[0]
    htb = ht_ref[0]
    mh = mh_ref[0]
    n = xb.shape[1]
    eye = (jax.lax.broadcasted_iota(jnp.int32, (TCH, TCH), 0)
           == jax.lax.broadcasted_iota(jnp.int32, (TCH, TCH), 1)
           ).astype(jnp.float32)
    mht = jnp.concatenate(
        [_t0(mh[k * TCH:(k + 1) * TCH], eye) for k in range(n // TCH)],
        axis=1)
    mrt = mht - htb
    gt = (
        jnp.dot(wga_ref[...], htb, preferred_element_type=jnp.float32)
        + jnp.dot(wgb_ref[...], mrt, preferred_element_type=jnp.float32)
        + bg_ref[...]
    )
    gt = jax.nn.gelu(gt)
    outt = (
        jnp.dot(w2_ref[...], gt, preferred_element_type=jnp.float32)
        + b2_ref[...] + xb
    )
    ft = jax.nn.gelu(
        jnp.dot(wf1_ref[...], outt, preferred_element_type=jnp.float32)
        + bf1_ref[...]
    )
    y_ref[0] = (
        jnp.dot(wf2_ref[...], ft, preferred_element_type=jnp.float32)
        + bf2_ref[...] + outt
    )


def kernel(x, W_fc1, b_fc1, g1, be1, W_g, b_g, gg, bg, W_fc2, b_fc2, g2, be2,
           Wf1, bf1, gf1, bef1, Wf2, bf2, gf2, bef2):
    B, C, H, W = x.shape
    N = H * W
    BN = B * N

    W1p = W_fc1 * g1[None, :]
    b1r = (b_fc1 * g1 + be1)[None, :]
    b1c = (b_fc1 * g1 + be1)[:, None]
    Wgp = W_g * gg[None, :]
    bgc = (b_g * gg + bg)[:, None]
    WgAT, WgBT = Wgp[:C].T, Wgp[C:].T
    W2T = (W_fc2 * g2[None, :]).T
    b2c = (b_fc2 * g2 + be2)[:, None]
    Wf1T = (Wf1 * gf1[None, :]).T
    bf1c = (bf1 * gf1 + bef1)[:, None]
    Wf2T = (Wf2 * gf2[None, :]).T
    bf2c = (bf2 * gf2 + bef2)[:, None]

    x3 = x.reshape(B, C, N)

    h, ht, sq = pl.pallas_call(
        _fc1_kernel,
        grid=(B,),
        in_specs=[
            pl.BlockSpec((1, C, N), lambda b: (b, 0, 0)),
            pl.BlockSpec((C, C), lambda b: (0, 0)),
            pl.BlockSpec((1, C), lambda b: (0, 0)),
            pl.BlockSpec((C, 1), lambda b: (0, 0)),
        ],
        out_specs=[
            pl.BlockSpec((1, N, C), lambda b: (b, 0, 0)),
            pl.BlockSpec((1, C, N), lambda b: (b, 0, 0)),
            pl.BlockSpec((1, 1, N), lambda b: (b, 0, 0)),
        ],
        out_shape=[
            jax.ShapeDtypeStruct((B, N, C), jnp.float32),
            jax.ShapeDtypeStruct((B, C, N), jnp.float32),
            jax.ShapeDtypeStruct((B, 1, N), jnp.float32),
        ],
    )(x3, W1p, b1r, b1c)

    idx = pl.pallas_call(
        _knn_kernel,
        grid=(B, N // RT),
        in_specs=[
            pl.BlockSpec((1, RT, C), lambda b, r: (b, r, 0)),
            pl.BlockSpec((1, C, N), lambda b, r: (b, 0, 0)),
            pl.BlockSpec((1, 1, N), lambda b, r: (b, 0, 0)),
        ],
        out_specs=pl.BlockSpec((1, RT, KNN), lambda b, r: (b, r, 0)),
        out_shape=jax.ShapeDtypeStruct((B, N, KNN), jnp.int32),
    )(h, ht, sq)

    mesh = plsc.VectorSubcoreMesh(core_axis_name="c", subcore_axis_name="s")
    maxh_rows = pl.kernel(
        _sc_gather_max,
        out_type=jax.ShapeDtypeStruct((BN, C), jnp.float32),
        mesh=mesh,
        scratch_types=[
            pltpu.VMEM((400, KNN), jnp.int32),
            pltpu.VMEM((KNN, 400), jnp.int32),
            pltpu.VMEM((2, KNN, CH, C), jnp.float32),
            pltpu.VMEM((CH, C), jnp.float32),
            pltpu.SemaphoreType.DMA,
            pltpu.SemaphoreType.DMA,
        ],
        compiler_params=pltpu.CompilerParams(
            use_tc_tiling_on_sc=False, needs_layout_passes=False),
    )(h.reshape(BN, C), idx.reshape(BN, KNN))

    y = pl.pallas_call(
        _tail_kernel,
        grid=(B,),
        in_specs=[
            pl.BlockSpec((1, C, N), lambda b: (b, 0, 0)),
            pl.BlockSpec((1, C, N), lambda b: (b, 0, 0)),
            pl.BlockSpec((1, N, C), lambda b: (b, 0, 0)),
            pl.BlockSpec((2 * C, C), lambda b: (0, 0)),
            pl.BlockSpec((2 * C, C), lambda b: (0, 0)),
            pl.BlockSpec((2 * C, 1), lambda b: (0, 0)),
            pl.BlockSpec((C, 2 * C), lambda b: (0, 0)),
            pl.BlockSpec((C, 1), lambda b: (0, 0)),
            pl.BlockSpec((4 * C, C), lambda b: (0, 0)),
            pl.BlockSpec((4 * C, 1), lambda b: (0, 0)),
            pl.BlockSpec((C, 4 * C), lambda b: (0, 0)),
            pl.BlockSpec((C, 1), lambda b: (0, 0)),
        ],
        out_specs=pl.BlockSpec((1, C, N), lambda b: (b, 0, 0)),
        out_shape=jax.ShapeDtypeStruct((B, C, N), jnp.float32),
    )(x3, ht, maxh_rows.reshape(B, N, C), WgAT, WgBT, bgc, W2T, b2c,
      Wf1T, bf1c, Wf2T, bf2c)

    return y.reshape(B, C, H, W)
